# Initial kernel scaffold; baseline (speedup 1.0000x reference)
import jax
import jax.numpy as jnp
from jax.experimental import pallas as pl


def kernel(x, params, edge_index_res10, edge_index_res9, edge_index_res8, edge_index_res7, edge_index_res6, edge_index_res5, c2p_res10, c2p_res9, c2p_res8, c2p_res7, c2p_res6):
    raise NotImplementedError("write your pallas kernel here")


# speedup vs baseline: 4.4913x; 4.4913x over previous
import functools

import jax
import jax.numpy as jnp
from jax import lax
from jax.experimental import pallas as pl
from jax.experimental.pallas import tpu as pltpu
from jax.experimental.pallas import tpu_sc as plsc

_HID = {10: 64, 9: 128, 8: 128, 7: 256, 6: 256, 5: 512}
_SIZES = {10: 117649, 9: 16807, 8: 2401, 7: 343, 6: 49, 5: 7}
_RES = [10, 9, 8, 7, 6, 5]

_NC, _NS, _L = 2, 16, 16
_NW = _NC * _NS
_DUMP = 512
_SPM_BUDGET = 4_400_000
_CNT_BUDGET = 2_400_000
_F32 = jnp.float32
_I32 = jnp.int32


def _cdiv(a, b):
    return -(-a // b)


def _linear(x, W, b, *, act=None, row_scale=None, rs_mode="post", x2=None,
            W2=None, nrows=None, br=512):
    n = x.shape[0] if nrows is None else nrows
    k = x.shape[1]
    do = W.shape[1]
    br = min(br, max(8, _cdiv(n, 8) * 8))
    grid = (_cdiv(n, br),)
    in_specs = [pl.BlockSpec((br, k), lambda i: (i, 0)),
                pl.BlockSpec((k, do), lambda i: (0, 0))]
    args = [x, W]
    has2 = x2 is not None
    if has2:
        k2 = x2.shape[1]
        in_specs += [pl.BlockSpec((br, k2), lambda i: (i, 0)),
                     pl.BlockSpec((k2, do), lambda i: (0, 0))]
        args += [x2, W2]
    hasrs = row_scale is not None
    if hasrs:
        in_specs.append(pl.BlockSpec((br, 1), lambda i: (i, 0)))
        args.append(row_scale)
    hasb = b is not None
    if hasb:
        in_specs.append(pl.BlockSpec((1, do), lambda i: (0, 0)))
        args.append(b.reshape(1, do))

    def body(*refs):
        it = iter(refs)
        xr = next(it)
        wr = next(it)
        x2v = w2v = None
        if has2:
            x2v = next(it)[...]
            w2v = next(it)[...]
        xv = xr[...]
        rsv = None
        if hasrs:
            rsv = next(it)[...]
            if rs_mode == "div":
                xv = xv / rsv
        acc = jnp.dot(xv, wr[...], preferred_element_type=_F32,
                      precision=lax.Precision.HIGHEST)
        if has2:
            acc = acc + jnp.dot(x2v, w2v, preferred_element_type=_F32,
                                precision=lax.Precision.HIGHEST)
        if hasrs and rs_mode == "post":
            acc = acc * rsv
        if hasb:
            acc = acc + next(it)[...]
        if act == "gelu":
            acc = jax.nn.gelu(acc)
        next(it)[...] = acc

    return pl.pallas_call(
        body, grid=grid, in_specs=in_specs,
        out_specs=pl.BlockSpec((br, do), lambda i: (i, 0)),
        out_shape=jax.ShapeDtypeStruct((n, do), _F32))(*args)


def _gn_mean(s, dinv, b, n, br=512):
    d = s.shape[1]
    br = min(br, max(8, _cdiv(n, 8) * 8))
    grid = (_cdiv(n, br),)

    def body(s_ref, dv_ref, b_ref, o_ref):
        i = pl.program_id(0)
        y = s_ref[...] * dv_ref[...] + b_ref[...]
        rows = lax.broadcasted_iota(_I32, (br, 1), 0) + i * br
        y = jnp.where(rows < n, y, 0.0)

        @pl.when(i == 0)
        def _():
            o_ref[...] = jnp.zeros_like(o_ref)

        o_ref[0:1, :] += jnp.sum(y, axis=0, keepdims=True)

    return pl.pallas_call(
        body, grid=grid,
        in_specs=[pl.BlockSpec((br, d), lambda i: (i, 0)),
                  pl.BlockSpec((br, 1), lambda i: (i, 0)),
                  pl.BlockSpec((1, d), lambda i: (0, 0))],
        out_specs=pl.BlockSpec((8, d), lambda i: (0, 0)),
        out_shape=jax.ShapeDtypeStruct((8, d), _F32))(s, dinv, b.reshape(1, d))


def _gn_var(s, dinv, b, mstats, ms, n, br=512):
    d = s.shape[1]
    br = min(br, max(8, _cdiv(n, 8) * 8))
    grid = (_cdiv(n, br),)

    def body(s_ref, dv_ref, b_ref, m_ref, ms_ref, o_ref):
        i = pl.program_id(0)
        y = s_ref[...] * dv_ref[...] + b_ref[...]
        xc = y - ms_ref[...] * (m_ref[0:1, :] / n)
        rows = lax.broadcasted_iota(_I32, (br, 1), 0) + i * br
        xc = jnp.where(rows < n, xc, 0.0)

        @pl.when(i == 0)
        def _():
            o_ref[...] = jnp.zeros_like(o_ref)

        o_ref[0:1, :] += jnp.sum(xc * xc, axis=0, keepdims=True)

    row = lambda a: a.reshape(1, d)
    return pl.pallas_call(
        body, grid=grid,
        in_specs=[pl.BlockSpec((br, d), lambda i: (i, 0)),
                  pl.BlockSpec((br, 1), lambda i: (i, 0)),
                  pl.BlockSpec((1, d), lambda i: (0, 0)),
                  pl.BlockSpec((8, d), lambda i: (0, 0)),
                  pl.BlockSpec((1, d), lambda i: (0, 0))],
        out_specs=pl.BlockSpec((8, d), lambda i: (0, 0)),
        out_shape=jax.ShapeDtypeStruct((8, d), _F32))(
            s, dinv, row(b), mstats, row(ms))


def _gn_apply(s, dinv, b, mstats, vstats, gm, bt, ms, n, eps=1e-5, br=512):
    d = s.shape[1]
    br = min(br, max(8, _cdiv(n, 8) * 8))
    grid = (_cdiv(n, br),)

    def body(s_ref, dv_ref, b_ref, m_ref, v_ref, gm_ref, bt_ref, ms_ref,
             o_ref):
        y = s_ref[...] * dv_ref[...] + b_ref[...]
        mean = m_ref[0:1, :] / n
        var = v_ref[0:1, :] / n
        xc = y - ms_ref[...] * mean
        o_ref[...] = jax.nn.gelu(
            gm_ref[...] * xc / jnp.sqrt(var + eps) + bt_ref[...])

    row = lambda a: a.reshape(1, d)
    return pl.pallas_call(
        body, grid=grid,
        in_specs=[pl.BlockSpec((br, d), lambda i: (i, 0)),
                  pl.BlockSpec((br, 1), lambda i: (i, 0)),
                  pl.BlockSpec((1, d), lambda i: (0, 0)),
                  pl.BlockSpec((8, d), lambda i: (0, 0)),
                  pl.BlockSpec((8, d), lambda i: (0, 0)),
                  pl.BlockSpec((1, d), lambda i: (0, 0)),
                  pl.BlockSpec((1, d), lambda i: (0, 0)),
                  pl.BlockSpec((1, d), lambda i: (0, 0))],
        out_specs=pl.BlockSpec((br, d), lambda i: (i, 0)),
        out_shape=jax.ShapeDtypeStruct((n, d), _F32))(
            s, dinv, row(b), mstats, vstats, row(gm), row(bt), row(ms))


def _inv_scale(cnts, mode, n, br=512):
    br = min(br, max(8, _cdiv(n, 8) * 8))
    grid = (_cdiv(n, br),)

    def body(c_ref, o_ref):
        c = c_ref[:, 0:1]
        if mode == "deg":
            o_ref[...] = jnp.where(c > 0, lax.rsqrt(c), 0.0)
        else:
            o_ref[...] = jnp.maximum(c, 1.0)

    return pl.pallas_call(
        body, grid=grid,
        in_specs=[pl.BlockSpec((br, _L), lambda i: (i, 0))],
        out_specs=pl.BlockSpec((br, 1), lambda i: (i, 0)),
        out_shape=jax.ShapeDtypeStruct((n, 1), _F32))(cnts)


def _sc_cfg(e, g, tab_bytes):
    budget = (8_100_000 - tab_bytes) // _NS - 24_000
    ch = 128 if e >= _NS * 128 else 16
    unit = _NS * ch * 8
    for ncalls in (1, 2, 3, 4, 6, 8):
        e_pad = _cdiv(e, ncalls * unit) * unit
        ids = 2 * (e_pad // _NS) * 4
        for nb in (4, 2, 1):
            vmem = ids + nb * ch * (g + 1) * 64 + 2 * nb * ch * 4
            if vmem <= budget:
                return ch, nb, e_pad, ncalls
    raise ValueError("no feasible edge chunking")


def _ceil_log2(n):
    sh = 0
    while (1 << sh) < n:
        sh += 1
    return sh


def _slab_cfg(n_dst, d):
    s = d // _L
    sh = max(3, _ceil_log2(n_dst))
    while ((1 << sh) + _DUMP) * 64 > _SPM_BUDGET:
        sh -= 1
    rows = (min(1 << sh, _cdiv(n_dst, 8) * 8) + _DUMP)
    gmax = _SPM_BUDGET // (rows * 64)
    g = 1
    for cand in range(1, s // 2 + 1):
        if s % cand == 0 and cand <= gmax:
            g = cand
    return sh, g, g * rows * 64


def _row_split(nrows):
    rpt = _cdiv(nrows, _NS * 8) * 8
    st_full = nrows // rpt
    rem = nrows - st_full * rpt
    return rpt, st_full, rem


def _mesh():
    return plsc.VectorSubcoreMesh(core_axis_name="c", subcore_axis_name="s",
                                  num_cores=_NC, num_subcores=_NS)


def _remap_store(didv, c, didx2v, b, sh, uu, n_dump_base):
    ch = didx2v.shape[-1]
    mask = (1 << sh) - 1
    for t in range(ch // _L):
        dv = didv[c, pl.ds(t * _L, _L)]
        u = lax.shift_right_logical(dv, sh)
        tloc = lax.bitwise_and(dv, mask)
        dump = n_dump_base + lax.bitwise_and(dv, _DUMP - 1)
        didx2v[b, pl.ds(t * _L, _L)] = jnp.where(u == uu, tloc, dump)


@functools.cache
def _make_scatter(n_src, n_dst, d, e_pad, ch, nb, sh, g):
    s = d // _L
    assert s % g == 0
    ec = e_pad // _NS
    nchunk = ec // ch
    nb = min(nb, nchunk)
    assert nchunk % nb == 0
    rng = 1 << sh
    nu_nodes = _cdiv(n_dst, rng)
    ngrp = s // g
    split = nu_nodes > 1
    rows_tab = (rng if split else _cdiv(n_dst, 8) * 8) + _DUMP
    dump_base = rng if split else n_dst

    scratch = ([pltpu.VMEM((nchunk, ch), _I32),
                pltpu.VMEM((nchunk, ch), _I32),
                pltpu.VMEM((nb, ch), _I32),
                pltpu.VMEM((g, nb, ch), _I32),
                pltpu.VMEM((g, nb, ch, _L), _F32)]
               + [pltpu.VMEM_SHARED((rows_tab, _L), _F32) for _ in range(g)]
               + [pltpu.SemaphoreType.DMA] * (2 * nb))

    def body(src_hbm, dst_hbm, tab_hbm, init_hbm, out_hbm, sidv, didv, didx2v,
             gidxv, bufv, *rest):
        accs = rest[:g]
        gsem = rest[g:g + nb]
        ssem = rest[g + nb:]
        cid = lax.axis_index("c")
        sid = lax.axis_index("s")
        pltpu.sync_copy(src_hbm.at[pl.ds(sid * nchunk, nchunk)], sidv)
        pltpu.sync_copy(dst_hbm.at[pl.ds(sid * nchunk, nchunk)], didv)

        def compute_gidx(c, b, k0):
            for t in range(ch // _L):
                sv = sidv[c, pl.ds(t * _L, _L)]
                for j in range(g):
                    gidxv[j, b, pl.ds(t * _L, _L)] = sv * s + (k0 + j)

        def start_gathers(b):
            for j in range(g):
                pltpu.async_copy(tab_hbm.at[gidxv.at[j, b]], bufv.at[j, b],
                                 gsem[b])

        def wait_gathers(b):
            for j in range(g):
                pltpu.make_async_copy(tab_hbm.at[gidxv.at[j, b]],
                                      bufv.at[j, b], gsem[b]).wait()

        def start_scatters(c, b):
            idx = didx2v.at[b] if split else didv.at[c]
            for j in range(g):
                pltpu.async_copy(bufv.at[j, b], accs[j].at[idx], ssem[b],
                                 add=True)

        def wait_scatters(b):
            idx = didx2v.at[b] if split else didv.at[0]
            for j in range(g):
                pltpu.make_async_copy(bufv.at[j, b], accs[j].at[idx],
                                      ssem[b]).wait()

        units = [(un, kg) for un in range(nu_nodes) for kg in range(ngrp)]
        for t, (un, kg) in enumerate(units):
            base = un * rng
            rows_real = min(rng, n_dst - base)
            k0 = kg * g
            rpt, st_full, rem = _row_split(rows_real)

            def tile_rows(fn):
                @pl.when(sid < st_full)
                def _():
                    fn(sid * rpt, rpt)
                if rem:
                    @pl.when(sid == st_full)
                    def _():
                        fn(st_full * rpt, rem)

            @pl.when(cid == (t % _NC))
            def _(un=un, base=base, k0=k0, tile_rows=tile_rows):
                for j in range(g):
                    tile_rows(lambda r0, nr, j=j: pltpu.sync_copy(
                        init_hbm.at[pl.ds(base + r0, nr),
                                    pl.ds((k0 + j) * _L, _L)],
                        accs[j].at[pl.ds(r0, nr), :]))
                plsc.subcore_barrier()

                for b in range(nb):
                    compute_gidx(b, b, k0)
                    start_gathers(b)

                def sup_body(i, _):
                    i0 = i * nb
                    for b in range(nb):
                        wait_gathers(b)
                        if split:
                            _remap_store(didv, i0 + b, didx2v, b, sh, un,
                                         rng)
                        start_scatters(i0 + b, b)
                    for b in range(nb):
                        c2 = i0 + b + nb

                        @pl.when(c2 < nchunk)
                        def _(c2=c2, b=b):
                            wait_scatters(b)
                            compute_gidx(c2, b, k0)
                            start_gathers(b)
                    return 0

                lax.fori_loop(0, nchunk // nb, sup_body, 0)
                for b in range(nb):
                    wait_scatters(b)
                plsc.subcore_barrier()

                for j in range(g):
                    tile_rows(lambda r0, nr, j=j: pltpu.sync_copy(
                        accs[j].at[pl.ds(r0, nr), :],
                        out_hbm.at[pl.ds(base + r0, nr),
                                   pl.ds((k0 + j) * _L, _L)]))
                plsc.subcore_barrier()

    return pl.kernel(
        body,
        out_type=jax.ShapeDtypeStruct((n_dst, d), _F32),
        mesh=_mesh(),
        scratch_types=scratch,
        compiler_params=pltpu.CompilerParams(use_tc_tiling_on_sc=False))


def _cnt_shift(n_dst):
    sh = max(3, _ceil_log2(n_dst))
    while ((1 << sh) + _DUMP) * 64 > _CNT_BUDGET:
        sh -= 1
    return sh


@functools.cache
def _make_counts(n_dst, e_pad, ch, nb, sh):
    ec = e_pad // _NS
    nchunk = ec // ch
    nb = min(nb, nchunk)
    assert nchunk % nb == 0
    rng = 1 << sh
    nunits = _cdiv(n_dst, rng)
    rows_tab = rng + _DUMP

    scratch = ([pltpu.VMEM((nchunk, ch), _I32),
                pltpu.VMEM((nb, ch), _I32),
                pltpu.VMEM((ch, _L), _F32),
                pltpu.VMEM_SHARED((rows_tab, _L), _F32)]
               + [pltpu.SemaphoreType.DMA] * nb)

    def body(dst_hbm, zinit_hbm, out_hbm, didv, didx2v, onesv, acc, *sems):
        cid = lax.axis_index("c")
        sid = lax.axis_index("s")
        pltpu.sync_copy(dst_hbm.at[pl.ds(sid * nchunk, nchunk)], didv)
        for t in range(ch):
            onesv[t, :] = jnp.ones((_L,), _F32)

        def start(b):
            pltpu.async_copy(onesv, acc.at[didx2v.at[b]], sems[b], add=True)

        def wait(b):
            pltpu.make_async_copy(onesv, acc.at[didx2v.at[0]], sems[b]).wait()

        for u in range(nunits):
            base = u * rng
            rows_real = min(rng, n_dst - base)
            rpt, st_full, rem = _row_split(rows_real)

            def tile_rows(fn):
                @pl.when(sid < st_full)
                def _():
                    fn(sid * rpt, rpt)
                if rem:
                    @pl.when(sid == st_full)
                    def _():
                        fn(st_full * rpt, rem)

            @pl.when(cid == (u % _NC))
            def _(u=u, base=base, tile_rows=tile_rows):
                tile_rows(lambda r0, nr: pltpu.sync_copy(
                    zinit_hbm.at[pl.ds(base + r0, nr), :],
                    acc.at[pl.ds(r0, nr), :]))
                plsc.subcore_barrier()

                def sup_body(i, _):
                    i0 = i * nb
                    for b in range(nb):
                        @pl.when(i > 0)
                        def _(b=b):
                            wait(b)
                        _remap_store(didv, i0 + b, didx2v, b, sh, u, rng)
                        start(b)
                    return 0

                lax.fori_loop(0, nchunk // nb, sup_body, 0)
                for b in range(nb):
                    wait(b)
                plsc.subcore_barrier()

                tile_rows(lambda r0, nr: pltpu.sync_copy(
                    acc.at[pl.ds(r0, nr), :],
                    out_hbm.at[pl.ds(base + r0, nr), :]))
                plsc.subcore_barrier()

    return pl.kernel(
        body,
        out_type=jax.ShapeDtypeStruct((n_dst, _L), _F32),
        mesh=_mesh(),
        scratch_types=scratch,
        compiler_params=pltpu.CompilerParams(use_tc_tiling_on_sc=False))


@functools.cache
def _make_gather(n_par, d_par, n_out_pad, ch, nb):
    ec = n_out_pad // _NW
    nchunk = ec // ch
    nb = min(nb, nchunk)
    assert nchunk % nb == 0

    scratch = ([pltpu.VMEM((nchunk, ch), _I32),
                pltpu.VMEM((nb, ch, d_par), _F32)]
               + [pltpu.SemaphoreType.DMA] * (2 * nb))

    def body(idx_hbm, tab_hbm, out_hbm, idxv, bufv, *sems):
        cid = lax.axis_index("c")
        sid = lax.axis_index("s")
        wid = sid * _NC + cid
        pltpu.sync_copy(idx_hbm.at[pl.ds(wid * nchunk, nchunk)], idxv)
        gsem = sems[:nb]
        wsem = sems[nb:]

        def start_gather(c, b):
            pltpu.async_copy(tab_hbm.at[idxv.at[c]], bufv.at[b], gsem[b])

        def wait_gather(b):
            pltpu.make_async_copy(tab_hbm.at[idxv.at[0]], bufv.at[b],
                                  gsem[b]).wait()

        def start_write(c, b):
            pltpu.async_copy(
                bufv.at[b],
                out_hbm.at[pl.ds((wid * nchunk + c) * ch, ch), :], wsem[b])

        def wait_write(b):
            pltpu.make_async_copy(
                bufv.at[b],
                out_hbm.at[pl.ds(wid * nchunk * ch, ch), :], wsem[b]).wait()

        for b in range(nb):
            start_gather(b, b)

        def sup_body(i, _):
            i0 = i * nb
            for b in range(nb):
                wait_gather(b)
                start_write(i0 + b, b)
            for b in range(nb):
                c2 = i0 + b + nb

                @pl.when(c2 < nchunk)
                def _(c2=c2, b=b):
                    wait_write(b)
                    start_gather(c2, b)
            return 0

        lax.fori_loop(0, nchunk // nb, sup_body, 0)
        for b in range(nb):
            wait_write(b)

    return pl.kernel(
        body,
        out_type=jax.ShapeDtypeStruct((n_out_pad, d_par), _F32),
        mesh=_mesh(),
        scratch_types=scratch)


def _pad_ids(ids, e_pad, n_dst, ch):
    pad = e_pad - ids.shape[0]
    if pad:
        fill = jnp.full((pad,), n_dst, _I32) + (
            jnp.arange(pad, dtype=_I32) % 8)
        ids = jnp.concatenate([ids, fill])
    return ids.reshape(e_pad // ch, ch)


def _pad_src(ids, e_pad, ch):
    pad = e_pad - ids.shape[0]
    if pad:
        ids = jnp.concatenate([ids, jnp.arange(pad, dtype=_I32) % 8])
    return ids.reshape(e_pad // ch, ch)


def _segsum(tab_src, init, parts, n_src, n_dst, d, e_pad, ch, nb, sh, g):
    k = _make_scatter(n_src, n_dst, d, e_pad, ch, nb, sh, g)
    tab = tab_src.reshape(n_src * (d // _L), _L)
    s = init
    for src2d, dst2d in parts:
        s = k(src2d, dst2d, tab, s)
    return s


def _gather_cfg(n_out):
    for ch, nb in ((128, 2), (32, 2), (8, 1)):
        unit = _NW * ch * 8
        n_pad = _cdiv(n_out, unit) * unit
        if n_pad - n_out <= max(unit // 4, n_out // 8):
            return ch, nb, n_pad
    return 8, 1, _cdiv(n_out, _NW * 8 * 8) * _NW * 8 * 8


def kernel(x, params, edge_index_res10, edge_index_res9, edge_index_res8,
           edge_index_res7, edge_index_res6, edge_index_res5, c2p_res10,
           c2p_res9, c2p_res8, c2p_res7, c2p_res6):
    eis = {10: edge_index_res10, 9: edge_index_res9, 8: edge_index_res8,
           7: edge_index_res7, 6: edge_index_res6, 5: edge_index_res5}
    c2ps = {10: c2p_res10, 9: c2p_res9, 8: c2p_res8, 7: c2p_res7,
            6: c2p_res6}

    lvl = {}
    for r in _RES:
        n, d = _SIZES[r], _HID[r]
        e = n * 7
        sh_s, g, tabb = _slab_cfg(n, d)
        ch, nb, e_pad, ncalls = _sc_cfg(e, g, tabb)
        e_tot = e_pad * ncalls
        sl = jnp.arange(n, dtype=_I32)
        src2d = _pad_src(jnp.concatenate([eis[r][0], sl]), e_tot, ch)
        dst2d = _pad_ids(jnp.concatenate([eis[r][1], sl]), e_tot, n, ch)
        rc = e_pad // ch
        parts = [(src2d[i * rc:(i + 1) * rc], dst2d[i * rc:(i + 1) * rc])
                 for i in range(ncalls)]
        cnt = _make_counts(n, e_tot, ch, nb, _cnt_shift(n))(
            dst2d, jnp.zeros((n, _L), _F32))[:, 0:1]
        dinv = jnp.where(cnt > 0, cnt ** -0.5, 0.0)
        lvl[r] = dict(n=n, d=d, parts=parts, e_pad=e_pad,
                      ch=ch, nb=nb, sh=sh_s, g=g, dinv=dinv,
                      z=jnp.zeros((n, d), _F32))
        if r > 5:
            npar = _SIZES[r - 1]
            shm, gm, tabbm = _slab_cfg(npar, d)
            chm, nbm, m_pad, mcalls = _sc_cfg(n, gm, tabbm)
            m_tot = m_pad * mcalls
            msrc = _pad_src(jnp.arange(n, dtype=_I32), m_tot, chm)
            mdst = _pad_ids(c2ps[r], m_tot, npar, chm)
            rcm = m_pad // chm
            mparts = [(msrc[i * rcm:(i + 1) * rcm],
                       mdst[i * rcm:(i + 1) * rcm]) for i in range(mcalls)]
            mcnt = _make_counts(npar, m_tot, chm, nbm, _cnt_shift(npar))(
                mdst, jnp.zeros((npar, _L), _F32))[:, 0:1]
            cinv = jnp.clip(mcnt, 1.0)
            chg, nbg, n_gpad = _gather_cfg(n)
            gidx = _pad_src(c2ps[r], n_gpad, chg)
            lvl[r].update(npar=npar, mparts=mparts, m_pad=m_pad,
                          chm=chm, nbm=nbm, shm=shm, gmm=gm, cinv=cinv,
                          gidx=gidx, chg=chg, nbg=nbg, n_gpad=n_gpad)

    def gcn_block(h, p, L):
        n, d = L["n"], L["d"]
        g = _linear(h, p["W"], None, row_scale=L["dinv"], rs_mode="post")
        s = _segsum(g, L["z"], L["parts"], n, n, d, L["e_pad"],
                    L["ch"], L["nb"], L["sh"], L["g"])
        ms_ = _gn_mean(s, L["dinv"], p["b"], n)
        vs_ = _gn_var(s, L["dinv"], p["b"], ms_, p["ms"], n)
        return _gn_apply(s, L["dinv"], p["b"], ms_, vs_, p["g"], p["bt"],
                         p["ms"], n)

    h = _linear(x, params["in"]["W"], params["in"]["b"], act="gelu")
    skips = {}
    for r in _RES:
        L = lvl[r]
        for p in params["enc%d" % r]:
            h = gcn_block(h, p, L)
        skips[r] = h
        if r > 5:
            n, d, npar = L["n"], L["d"], L["npar"]
            s = _segsum(h, jnp.zeros((npar, d), _F32), L["mparts"],
                        n, npar, d, L["m_pad"], L["chm"], L["nbm"],
                        L["shm"], L["gmm"])
            tp = params["enc_tr%d" % r]
            h = _linear(s, tp["W"], tp["b"], act="gelu",
                        row_scale=L["cinv"], rs_mode="div")

    for r in _RES[::-1]:
        L = lvl[r]
        if r > 5:
            n, dpar = L["n"], _HID[r - 1]
            bc = _make_gather(L["npar"], dpar, L["n_gpad"], L["chg"],
                              L["nbg"])(L["gidx"], h)
            tp = params["dec_tr%d" % r]
            W1 = tp["W"][:dpar]
            W2 = tp["W"][dpar:]
            h = _linear(bc, W1, tp["b"], act="gelu", x2=skips[r], W2=W2,
                        nrows=n)
        for p in params["dec%d" % r]:
            h = gcn_block(h, p, L)

    return _linear(h, params["out"]["W"], params["out"]["b"])
